# baseline (speedup 1.0000x reference)
import functools

import jax
import jax.numpy as jnp
from jax import lax
from jax.experimental import pallas as pl
from jax.experimental.pallas import tpu as pltpu, tpu_sc as plsc

_D = 32
_V = 8192
_TN = 1024
_TM = 8192


def _cbn_body(cb_ref, cbn_ref):
    c = cb_ref[...]
    n = jnp.sqrt(jnp.sum(c * c, axis=1, keepdims=True))
    cbn_ref[...] = (c / jnp.maximum(n, 1e-12)).astype(jnp.bfloat16)


def _tc_cbn(cb):
    return pl.pallas_call(
        _cbn_body,
        out_shape=jax.ShapeDtypeStruct((_V, _D), jnp.bfloat16),
    )(cb)


def _vq_body(x_ref, cbn_ref, idx_ref):
    x = x_ref[...]
    xn = jnp.sqrt(jnp.sum(x * x, axis=1, keepdims=True))
    xb = (x / jnp.maximum(xn, 1e-12)).astype(jnp.bfloat16)
    s = lax.dot_general(
        xb, cbn_ref[...], (((1,), (1,)), ((), ())),
        preferred_element_type=jnp.float32,
    )
    a = jnp.argmax(s, axis=1)
    idx_ref[0, 0, :] = a.astype(jnp.int32)


def _tc_vq(x2, cbn):
    n_tok = x2.shape[0]
    nblk = n_tok // _TN
    idx3 = pl.pallas_call(
        _vq_body,
        grid=(nblk,),
        in_specs=[
            pl.BlockSpec((_TN, _D), lambda i: (i, 0)),
            pl.BlockSpec((_V, _D), lambda i: (0, 0)),
        ],
        out_specs=pl.BlockSpec((1, 1, _TN), lambda i: (i, 0, 0)),
        out_shape=jax.ShapeDtypeStruct((nblk, 1, _TN), jnp.int32),
        compiler_params=pltpu.CompilerParams(
            dimension_semantics=("parallel",),
        ),
    )(x2, cbn)
    return idx3.reshape(n_tok)


def _mean_body(x_ref, zq_ref, out_ref, acc_ref):
    i = pl.program_id(0)
    nblk = pl.num_programs(0)

    @pl.when(i == 0)
    def _init():
        acc_ref[0] = 0.0

    x = x_ref[...]
    z = zq_ref[...]
    ones = jnp.ones((_D, 128), jnp.float32)
    dn = (((1,), (0,)), ((), ()))
    sxz = lax.dot_general(x * z, ones, dn, preferred_element_type=jnp.float32)[:, 0]
    sxx = lax.dot_general(x * x, ones, dn, preferred_element_type=jnp.float32)[:, 0]
    szz = lax.dot_general(z * z, ones, dn, preferred_element_type=jnp.float32)[:, 0]
    xinv = 1.0 / jnp.maximum(jnp.sqrt(sxx), 1e-12)
    zinv = 1.0 / jnp.maximum(jnp.sqrt(szz), 1e-12)
    acc_ref[0] += jnp.sum(sxz * xinv * zinv)

    @pl.when(i == nblk - 1)
    def _fin():
        out_ref[0, 0] = acc_ref[0] / (nblk * _TM)


def _tc_mean(x2, zq):
    n_tok = x2.shape[0]
    nblk = n_tok // _TM
    out = pl.pallas_call(
        _mean_body,
        grid=(nblk,),
        in_specs=[
            pl.BlockSpec((_TM, _D), lambda i: (i, 0)),
            pl.BlockSpec((_TM, _D), lambda i: (i, 0)),
        ],
        out_specs=pl.BlockSpec(memory_space=pltpu.SMEM),
        out_shape=jax.ShapeDtypeStruct((1, 1), jnp.float32),
        scratch_shapes=[pltpu.SMEM((1,), jnp.float32)],
        compiler_params=pltpu.CompilerParams(
            dimension_semantics=("arbitrary",),
        ),
    )(x2, zq)
    return out[0, 0]


def _sc_gather(table, idx):
    n_tok = idx.shape[0]
    info = plsc.get_sparse_core_info()
    nc, ns = info.num_cores, info.num_subcores
    nw = nc * ns
    bpw = n_tok // nw
    mesh = plsc.VectorSubcoreMesh(core_axis_name="c", subcore_axis_name="s")

    @functools.partial(
        pl.kernel, mesh=mesh,
        out_type=jax.ShapeDtypeStruct((n_tok, _D), jnp.float32),
        scratch_types=[
            pltpu.VMEM((bpw,), jnp.int32),
            pltpu.VMEM((bpw, _D), jnp.float32),
            pltpu.SemaphoreType.DMA,
        ],
        compiler_params=pltpu.CompilerParams(use_tc_tiling_on_sc=False),
    )
    def k(table_hbm, idx_hbm, out_hbm, idx_v, rows_v, sem):
        wid = lax.axis_index("s") * nc + lax.axis_index("c")
        base = wid * bpw
        pltpu.sync_copy(idx_hbm.at[pl.ds(base, bpw)], idx_v)
        pltpu.async_copy(table_hbm.at[idx_v], rows_v, sem).wait()
        pltpu.sync_copy(rows_v, out_hbm.at[pl.ds(base, bpw)])

    return k(table, idx)


def kernel(x, codebook):
    b, t, d = x.shape
    x2 = x.reshape(-1, d)
    cbn = _tc_cbn(codebook)
    nch = 4
    csz = x2.shape[0] // nch
    idxs, zqs, means = [], [], []
    for k in range(nch):
        xk = lax.slice_in_dim(x2, k * csz, (k + 1) * csz, axis=0)
        ik = _tc_vq(xk, cbn)
        zk = _sc_gather(codebook, ik)
        mk = _tc_mean(xk, zk)
        idxs.append(ik)
        zqs.append(zk)
        means.append(mk)
    idx_flat = jnp.concatenate(idxs)
    z_q = jnp.concatenate(zqs)
    mean_sim = jnp.stack(means).mean()
    return z_q.reshape(b, t, d), idx_flat.reshape(b, t), mean_sim


# speedup vs baseline: 1.0431x; 1.0431x over previous
import functools

import jax
import jax.numpy as jnp
from jax import lax
from jax.experimental import pallas as pl
from jax.experimental.pallas import tpu as pltpu, tpu_sc as plsc

_D = 32
_V = 8192
_TN = 512
_TM = 8192


def _cbn_body(cb_ref, cbn_ref):
    c = cb_ref[...]
    n = jnp.sqrt(jnp.sum(c * c, axis=1, keepdims=True))
    cbn_ref[...] = (c / jnp.maximum(n, 1e-12)).astype(jnp.bfloat16)


def _tc_cbn(cb):
    return pl.pallas_call(
        _cbn_body,
        out_shape=jax.ShapeDtypeStruct((_V, _D), jnp.bfloat16),
    )(cb)


def _vq_body(x_ref, cbn_ref, idx_ref, msum_ref, acc_ref):
    i = pl.program_id(0)
    nblk = pl.num_programs(0)

    @pl.when(i == 0)
    def _init():
        acc_ref[0] = 0.0

    x = x_ref[...]
    xn = jnp.sqrt(jnp.sum(x * x, axis=1, keepdims=True))
    xb = (x / jnp.maximum(xn, 1e-12)).astype(jnp.bfloat16)
    s = lax.dot_general(
        xb, cbn_ref[...], (((1,), (1,)), ((), ())),
        preferred_element_type=jnp.float32,
    )
    a = jnp.argmax(s, axis=1)
    idx_ref[0, 0, :] = a.astype(jnp.int32)
    acc_ref[0] += jnp.sum(jnp.max(s, axis=1))

    @pl.when(i == nblk - 1)
    def _fin():
        msum_ref[0, 0] = acc_ref[0] / (nblk * _TN)


def _tc_vq(x2, cbn):
    n_tok = x2.shape[0]
    nblk = n_tok // _TN
    idx3, msum = pl.pallas_call(
        _vq_body,
        grid=(nblk,),
        in_specs=[
            pl.BlockSpec((_TN, _D), lambda i: (i, 0)),
            pl.BlockSpec((_V, _D), lambda i: (0, 0)),
        ],
        out_specs=[
            pl.BlockSpec((1, 1, _TN), lambda i: (i, 0, 0)),
            pl.BlockSpec(memory_space=pltpu.SMEM),
        ],
        out_shape=[
            jax.ShapeDtypeStruct((nblk, 1, _TN), jnp.int32),
            jax.ShapeDtypeStruct((1, 1), jnp.float32),
        ],
        scratch_shapes=[pltpu.SMEM((1,), jnp.float32)],
        compiler_params=pltpu.CompilerParams(
            dimension_semantics=("arbitrary",),
        ),
    )(x2, cbn)
    return idx3.reshape(n_tok), msum[0, 0]


def _mean_body(x_ref, zq_ref, out_ref, acc_ref):
    i = pl.program_id(0)
    nblk = pl.num_programs(0)

    @pl.when(i == 0)
    def _init():
        acc_ref[0] = 0.0

    x = x_ref[...]
    z = zq_ref[...]
    ones = jnp.ones((_D, 128), jnp.float32)
    dn = (((1,), (0,)), ((), ()))
    sxz = lax.dot_general(x * z, ones, dn, preferred_element_type=jnp.float32)[:, 0]
    sxx = lax.dot_general(x * x, ones, dn, preferred_element_type=jnp.float32)[:, 0]
    szz = lax.dot_general(z * z, ones, dn, preferred_element_type=jnp.float32)[:, 0]
    xinv = 1.0 / jnp.maximum(jnp.sqrt(sxx), 1e-12)
    zinv = 1.0 / jnp.maximum(jnp.sqrt(szz), 1e-12)
    acc_ref[0] += jnp.sum(sxz * xinv * zinv)

    @pl.when(i == nblk - 1)
    def _fin():
        out_ref[0, 0] = acc_ref[0] / (nblk * _TM)


def _tc_mean(x2, zq):
    n_tok = x2.shape[0]
    nblk = n_tok // _TM
    out = pl.pallas_call(
        _mean_body,
        grid=(nblk,),
        in_specs=[
            pl.BlockSpec((_TM, _D), lambda i: (i, 0)),
            pl.BlockSpec((_TM, _D), lambda i: (i, 0)),
        ],
        out_specs=pl.BlockSpec(memory_space=pltpu.SMEM),
        out_shape=jax.ShapeDtypeStruct((1, 1), jnp.float32),
        scratch_shapes=[pltpu.SMEM((1,), jnp.float32)],
        compiler_params=pltpu.CompilerParams(
            dimension_semantics=("arbitrary",),
        ),
    )(x2, zq)
    return out[0, 0]


def _sc_gather(table, idx):
    n_tok = idx.shape[0]
    info = plsc.get_sparse_core_info()
    nc, ns = info.num_cores, info.num_subcores
    nw = nc * ns
    bpw = n_tok // nw
    mesh = plsc.VectorSubcoreMesh(core_axis_name="c", subcore_axis_name="s")

    @functools.partial(
        pl.kernel, mesh=mesh,
        out_type=jax.ShapeDtypeStruct((n_tok, _D), jnp.float32),
        scratch_types=[
            pltpu.VMEM((bpw,), jnp.int32),
            pltpu.VMEM((bpw, _D), jnp.float32),
            pltpu.SemaphoreType.DMA,
        ],
        compiler_params=pltpu.CompilerParams(use_tc_tiling_on_sc=False),
    )
    def k(table_hbm, idx_hbm, out_hbm, idx_v, rows_v, sem):
        wid = lax.axis_index("s") * nc + lax.axis_index("c")
        base = wid * bpw
        pltpu.sync_copy(idx_hbm.at[pl.ds(base, bpw)], idx_v)
        pltpu.async_copy(table_hbm.at[idx_v], rows_v, sem).wait()
        pltpu.sync_copy(rows_v, out_hbm.at[pl.ds(base, bpw)])

    return k(table, idx)


def kernel(x, codebook):
    b, t, d = x.shape
    x2 = x.reshape(-1, d)
    cbn = _tc_cbn(codebook)
    idx_flat, mean_sim = _tc_vq(x2, cbn)
    z_q = _sc_gather(codebook, idx_flat)
    return z_q.reshape(b, t, d), idx_flat.reshape(b, t), mean_sim


# speedup vs baseline: 1.1878x; 1.1387x over previous
import functools

import jax
import jax.numpy as jnp
from jax import lax
from jax.experimental import pallas as pl
from jax.experimental.pallas import tpu as pltpu, tpu_sc as plsc

_D = 32
_V = 8192
_TN = 1024
_TM = 8192


def _cbn_body(cb_ref, cbn_ref):
    c = cb_ref[...]
    n = jnp.sqrt(jnp.sum(c * c, axis=1, keepdims=True))
    cbn_ref[...] = (c / jnp.maximum(n, 1e-12)).astype(jnp.bfloat16)


def _tc_cbn(cb):
    return pl.pallas_call(
        _cbn_body,
        out_shape=jax.ShapeDtypeStruct((_V, _D), jnp.bfloat16),
    )(cb)


def _vq_body(x_ref, cbn_ref, idx_ref):
    x = x_ref[...]
    xn = jnp.sqrt(jnp.sum(x * x, axis=1, keepdims=True))
    xb = (x / jnp.maximum(xn, 1e-12)).astype(jnp.bfloat16)
    s = lax.dot_general(
        xb, cbn_ref[...], (((1,), (1,)), ((), ())),
        preferred_element_type=jnp.float32,
    )
    a = jnp.argmax(s, axis=1)
    idx_ref[...] = a.astype(jnp.int32).reshape(_TN // 128, 128)


def _tc_vq(x2, cbn):
    n_tok = x2.shape[0]
    nblk = n_tok // _TN
    idx3 = pl.pallas_call(
        _vq_body,
        grid=(nblk,),
        in_specs=[
            pl.BlockSpec((_TN, _D), lambda i: (i, 0)),
            pl.BlockSpec((_V, _D), lambda i: (0, 0)),
        ],
        out_specs=pl.BlockSpec((_TN // 128, 128), lambda i: (i, 0)),
        out_shape=jax.ShapeDtypeStruct((nblk * (_TN // 128), 128), jnp.int32),
        compiler_params=pltpu.CompilerParams(
            dimension_semantics=("parallel",),
        ),
    )(x2, cbn)
    return idx3.reshape(n_tok)


def _mean_body(x_ref, zq_ref, out_ref, acc_ref):
    i = pl.program_id(0)
    nblk = pl.num_programs(0)

    @pl.when(i == 0)
    def _init():
        acc_ref[0] = 0.0

    x = x_ref[...]
    z = zq_ref[...]
    ones = jnp.ones((_D, 128), jnp.float32)
    dn = (((1,), (0,)), ((), ()))
    sxz = lax.dot_general(x * z, ones, dn, preferred_element_type=jnp.float32)[:, 0]
    sxx = lax.dot_general(x * x, ones, dn, preferred_element_type=jnp.float32)[:, 0]
    szz = lax.dot_general(z * z, ones, dn, preferred_element_type=jnp.float32)[:, 0]
    xinv = 1.0 / jnp.maximum(jnp.sqrt(sxx), 1e-12)
    zinv = 1.0 / jnp.maximum(jnp.sqrt(szz), 1e-12)
    acc_ref[0] += jnp.sum(sxz * xinv * zinv)

    @pl.when(i == nblk - 1)
    def _fin():
        out_ref[0, 0] = acc_ref[0] / (nblk * _TM)


def _tc_mean(x2, zq):
    n_tok = x2.shape[0]
    nblk = n_tok // _TM
    out = pl.pallas_call(
        _mean_body,
        grid=(nblk,),
        in_specs=[
            pl.BlockSpec((_TM, _D), lambda i: (i, 0)),
            pl.BlockSpec((_TM, _D), lambda i: (i, 0)),
        ],
        out_specs=pl.BlockSpec(memory_space=pltpu.SMEM),
        out_shape=jax.ShapeDtypeStruct((1, 1), jnp.float32),
        scratch_shapes=[pltpu.SMEM((1,), jnp.float32)],
        compiler_params=pltpu.CompilerParams(
            dimension_semantics=("arbitrary",),
        ),
    )(x2, zq)
    return out[0, 0]


def _sc_gather(table, idx):
    n_tok = idx.shape[0]
    info = plsc.get_sparse_core_info()
    nc, ns = info.num_cores, info.num_subcores
    nw = nc * ns
    bpw = n_tok // nw
    mesh = plsc.VectorSubcoreMesh(core_axis_name="c", subcore_axis_name="s")

    @functools.partial(
        pl.kernel, mesh=mesh,
        out_type=jax.ShapeDtypeStruct((n_tok, _D), jnp.float32),
        scratch_types=[
            pltpu.VMEM((bpw,), jnp.int32),
            pltpu.VMEM((bpw, _D), jnp.float32),
            pltpu.SemaphoreType.DMA,
        ],
        compiler_params=pltpu.CompilerParams(use_tc_tiling_on_sc=False),
    )
    def k(table_hbm, idx_hbm, out_hbm, idx_v, rows_v, sem):
        wid = lax.axis_index("s") * nc + lax.axis_index("c")
        base = wid * bpw
        pltpu.sync_copy(idx_hbm.at[pl.ds(base, bpw)], idx_v)
        pltpu.async_copy(table_hbm.at[idx_v], rows_v, sem).wait()
        pltpu.sync_copy(rows_v, out_hbm.at[pl.ds(base, bpw)])

    return k(table, idx)


def kernel(x, codebook):
    b, t, d = x.shape
    x2 = x.reshape(-1, d)
    cbn = _tc_cbn(codebook)
    idx_flat = _tc_vq(x2, cbn)
    z_q = _sc_gather(codebook, idx_flat)
    mean_sim = _tc_mean(x2, z_q)
    return z_q.reshape(b, t, d), idx_flat.reshape(b, t), mean_sim
